# baseline (speedup 1.0000x reference)
import functools
import math

import jax
import jax.numpy as jnp
from jax import lax
from jax.experimental import pallas as pl
from jax.experimental.pallas import tpu as pltpu
from jax.experimental.pallas import tpu_sc as plsc

EMBED = 784
SCALE = math.sqrt(EMBED)
VOCAB = 100000
NBATCH = 4096
NSEQ = 50

NC, NS = 2, 16
NW = NC * NS

B = NBATCH * NSEQ
BPW = B // NW
CHUNK = 32
NSLOT = 4
NCHUNK = BPW // CHUNK
NQUAD = NCHUNK // NSLOT

VBLOCK = 1024
IBLOCK = 512


def _scale_t_body(t_ref, o_ref):
    o_ref[...] = jnp.transpose(t_ref[...]) * SCALE


def _scale_transpose_table(table):
    t_t = jnp.transpose(table)
    return pl.pallas_call(
        _scale_t_body,
        grid=(pl.cdiv(VOCAB, VBLOCK),),
        in_specs=[pl.BlockSpec((EMBED, VBLOCK), lambda i: (0, i))],
        out_specs=pl.BlockSpec((VBLOCK, EMBED), lambda i: (i, 0)),
        out_shape=jax.ShapeDtypeStruct((VOCAB, EMBED), jnp.float32),
    )(t_t)


def _out_t_body(i_ref, o_ref):
    o_ref[...] = jnp.transpose(i_ref[...], (0, 2, 1))


def _transpose_out(inter):
    i3 = inter.reshape(NSEQ, NBATCH, EMBED)
    return pl.pallas_call(
        _out_t_body,
        grid=(NSEQ, NBATCH // IBLOCK),
        in_specs=[pl.BlockSpec((1, IBLOCK, EMBED), lambda s, i: (s, i, 0))],
        out_specs=pl.BlockSpec((1, EMBED, IBLOCK), lambda s, i: (s, 0, i)),
        out_shape=jax.ShapeDtypeStruct((NSEQ, EMBED, NBATCH), jnp.float32),
    )(i3)


def _make_gather_kernel():
    mesh = plsc.VectorSubcoreMesh(core_axis_name="c", subcore_axis_name="s")

    @functools.partial(
        pl.kernel,
        out_type=jax.ShapeDtypeStruct((B, EMBED), jnp.float32),
        mesh=mesh,
        compiler_params=pltpu.CompilerParams(use_tc_tiling_on_sc=False),
        scratch_types=[
            pltpu.VMEM((BPW,), jnp.int32),
            pltpu.VMEM((NSLOT, CHUNK, EMBED), jnp.float32),
        ]
        + [pltpu.SemaphoreType.DMA] * (2 * NSLOT),
    )
    def body(table_hbm, idx_hbm, out_hbm, idx_v, buf_v, *sems):
        sem_g = sems[:NSLOT]
        sem_s = sems[NSLOT:]
        wid = lax.axis_index("s") * NC + lax.axis_index("c")
        base = wid * BPW

        pltpu.sync_copy(idx_hbm.at[pl.ds(base, BPW)], idx_v)

        def gather_start(c, b):
            pltpu.async_copy(
                table_hbm.at[idx_v.at[pl.ds(c * CHUNK, CHUNK)]],
                buf_v.at[b], sem_g[b])

        def gather_wait(c, b):
            pltpu.make_async_copy(
                table_hbm.at[idx_v.at[pl.ds(c * CHUNK, CHUNK)]],
                buf_v.at[b], sem_g[b]).wait()

        def scatter_start(c, b):
            pltpu.async_copy(
                buf_v.at[b],
                out_hbm.at[pl.ds(base + c * CHUNK, CHUNK)], sem_s[b])

        def scatter_wait(c, b):
            pltpu.make_async_copy(
                buf_v.at[b],
                out_hbm.at[pl.ds(base + c * CHUNK, CHUNK)], sem_s[b]).wait()

        for s in range(NSLOT):
            gather_start(s, s)
        for b in range(NSLOT - 1):
            gather_wait(b, b)
            scatter_start(b, b)
        gather_wait(3, 3)
        scatter_start(3, 3)
        scatter_wait(0, 0)
        gather_start(4, 0)

        def quad(q, _):
            c0 = q * NSLOT
            for b in range(NSLOT):
                c = c0 + b
                nb = (b + 1) % NSLOT
                gather_wait(c, b)
                scatter_start(c, b)
                scatter_wait(c - 3, nb)
                gather_start(c + 1, nb)
            return _

        lax.fori_loop(1, NQUAD - 1, quad, 0)

        c0 = (NQUAD - 1) * NSLOT
        for b in range(NSLOT - 1):
            c = c0 + b
            gather_wait(c, b)
            scatter_start(c, b)
            scatter_wait(c - 3, b + 1)
            gather_start(c + 1, b + 1)
        gather_wait(c0 + 3, 3)
        scatter_start(c0 + 3, 3)

        for b in range(NSLOT):
            scatter_wait(c0 + b, b)

    return body


_gather_kernel = _make_gather_kernel()


@jax.jit
def kernel(x, table):
    idx = jnp.transpose(x).reshape(-1).astype(jnp.int32)
    scaled = table * SCALE
    inter = _gather_kernel(scaled, idx)
    return jnp.transpose(inter.reshape(NSEQ, NBATCH, EMBED), (1, 0, 2))


# speedup vs baseline: 1.5312x; 1.5312x over previous
import functools
import math

import jax
import jax.numpy as jnp
from jax import lax
from jax.experimental import pallas as pl
from jax.experimental.pallas import tpu as pltpu
from jax.experimental.pallas import tpu_sc as plsc

EMBED = 784
SCALE = math.sqrt(EMBED)
VOCAB = 100000
NBATCH = 4096
NSEQ = 50

NC, NS = 2, 16
NW = NC * NS

B = NBATCH * NSEQ
BPW = B // NW
CHUNK = 32
NSLOT = 4
NCHUNK = BPW // CHUNK
NQUAD = NCHUNK // NSLOT

VBLOCK = 4096
IBLOCK = 512


def _scale_t_body(t_ref, o_ref):
    o_ref[...] = jnp.transpose(t_ref[...]) * SCALE


def _scale_transpose_table(table):
    t_t = jnp.transpose(table)
    return pl.pallas_call(
        _scale_t_body,
        grid=(pl.cdiv(VOCAB, VBLOCK),),
        in_specs=[pl.BlockSpec((EMBED, VBLOCK), lambda i: (0, i))],
        out_specs=pl.BlockSpec((VBLOCK, EMBED), lambda i: (i, 0)),
        out_shape=jax.ShapeDtypeStruct((VOCAB, EMBED), jnp.float32),
    )(t_t)


def _out_t_body(i_ref, o_ref):
    o_ref[...] = jnp.transpose(i_ref[...], (0, 2, 1))


def _transpose_out(inter):
    i3 = inter.reshape(NSEQ, NBATCH, EMBED)
    return pl.pallas_call(
        _out_t_body,
        grid=(NSEQ, NBATCH // IBLOCK),
        in_specs=[pl.BlockSpec((1, IBLOCK, EMBED), lambda s, i: (s, i, 0))],
        out_specs=pl.BlockSpec((1, EMBED, IBLOCK), lambda s, i: (s, 0, i)),
        out_shape=jax.ShapeDtypeStruct((NSEQ, EMBED, NBATCH), jnp.float32),
    )(i3)


def _make_gather_kernel():
    mesh = plsc.VectorSubcoreMesh(core_axis_name="c", subcore_axis_name="s")

    @functools.partial(
        pl.kernel,
        out_type=jax.ShapeDtypeStruct((B, EMBED), jnp.float32),
        mesh=mesh,
        compiler_params=pltpu.CompilerParams(use_tc_tiling_on_sc=False),
        scratch_types=[
            pltpu.VMEM((BPW,), jnp.int32),
            pltpu.VMEM((NSLOT, CHUNK, EMBED), jnp.float32),
        ]
        + [pltpu.SemaphoreType.DMA] * (2 * NSLOT),
    )
    def body(table_hbm, idx_hbm, out_hbm, idx_v, buf_v, *sems):
        sem_g = sems[:NSLOT]
        sem_s = sems[NSLOT:]
        wid = lax.axis_index("s") * NC + lax.axis_index("c")
        base = wid * BPW

        pltpu.sync_copy(idx_hbm.at[pl.ds(base, BPW)], idx_v)

        def gather_start(c, b):
            pltpu.async_copy(
                table_hbm.at[idx_v.at[pl.ds(c * CHUNK, CHUNK)]],
                buf_v.at[b], sem_g[b])

        def gather_wait(c, b):
            pltpu.make_async_copy(
                table_hbm.at[idx_v.at[pl.ds(c * CHUNK, CHUNK)]],
                buf_v.at[b], sem_g[b]).wait()

        def scatter_start(c, b):
            pltpu.async_copy(
                buf_v.at[b],
                out_hbm.at[pl.ds(base + c * CHUNK, CHUNK)], sem_s[b])

        def scatter_wait(c, b):
            pltpu.make_async_copy(
                buf_v.at[b],
                out_hbm.at[pl.ds(base + c * CHUNK, CHUNK)], sem_s[b]).wait()

        for s in range(NSLOT):
            gather_start(s, s)
        for b in range(NSLOT - 1):
            gather_wait(b, b)
            scatter_start(b, b)
        gather_wait(3, 3)
        scatter_start(3, 3)
        scatter_wait(0, 0)
        gather_start(4, 0)

        def quad(q, _):
            c0 = q * NSLOT
            for b in range(NSLOT):
                c = c0 + b
                nb = (b + 1) % NSLOT
                gather_wait(c, b)
                scatter_start(c, b)
                scatter_wait(c - 3, nb)
                gather_start(c + 1, nb)
            return _

        lax.fori_loop(1, NQUAD - 1, quad, 0)

        c0 = (NQUAD - 1) * NSLOT
        for b in range(NSLOT - 1):
            c = c0 + b
            gather_wait(c, b)
            scatter_start(c, b)
            scatter_wait(c - 3, b + 1)
            gather_start(c + 1, b + 1)
        gather_wait(c0 + 3, 3)
        scatter_start(c0 + 3, 3)

        for b in range(NSLOT):
            scatter_wait(c0 + b, b)

    return body


_gather_kernel = _make_gather_kernel()


@jax.jit
def kernel(x, table):
    idx = jnp.transpose(x).reshape(-1).astype(jnp.int32)
    scaled = _scale_transpose_table(table)
    inter = _gather_kernel(scaled, idx)
    return jnp.transpose(inter.reshape(NSEQ, NBATCH, EMBED), (1, 0, 2))
